# Initial kernel scaffold; baseline (speedup 1.0000x reference)
import jax
import jax.numpy as jnp
from jax.experimental import pallas as pl


def kernel(x, edge_index, edge_attr, W1, Wr1, We1, W2, Wr2, We2):
    raise NotImplementedError("write your pallas kernel here")


# speedup vs baseline: 2.5695x; 2.5695x over previous
import functools

import jax
import jax.numpy as jnp
from jax import lax
from jax.experimental import pallas as pl
from jax.experimental.pallas import tpu as pltpu
from jax.experimental.pallas import tpu_sc as plsc

NC = 2
NS = 16
D = 128
DE = 16
CHUNK = 80


@functools.lru_cache(maxsize=None)
def _make_sc_pass(n_table, n_edges, npad):
  per_tile = n_edges // NS
  iters = per_tile // CHUNK
  assert per_tile % CHUNK == 0
  rows_per_tile = npad // NS

  mesh = plsc.VectorSubcoreMesh(core_axis_name="c", subcore_axis_name="s")

  out_type = [
      jax.ShapeDtypeStruct((npad, D), jnp.float32),
      jax.ShapeDtypeStruct((npad, D), jnp.float32),
  ]
  scratch = [
      pltpu.VMEM((CHUNK,), jnp.int32),
      pltpu.VMEM((CHUNK,), jnp.int32),
      pltpu.VMEM((CHUNK, D), jnp.float32),
      pltpu.VMEM((CHUNK, DE), jnp.float32),
      pltpu.VMEM((16, D), jnp.float32),
      pltpu.VMEM_SHARED((npad, D), jnp.float32),
  ]

  def body(x_hbm, feat_hbm, src_hbm, dst_hbm, init_hbm, outg, outh,
           sidx, didx, rows, featv, zg, acc):
    c = lax.axis_index("c")
    s = lax.axis_index("s")

    zvec = jnp.zeros((16,), jnp.float32)
    for r in range(16):
      for q in range(D // 16):
        zg[r, pl.ds(q * 16, 16)] = zvec

    base_r = s * rows_per_tile

    @pl.loop(0, rows_per_tile // 16)
    def _zero(i):
      pltpu.sync_copy(zg, acc.at[pl.ds(base_r + i * 16, 16)])

    ebase = s * per_tile

    @pl.when(c == 0)
    def _g_core():
      pltpu.sync_copy(src_hbm.at[pl.ds(ebase, CHUNK)], sidx)
      pltpu.sync_copy(dst_hbm.at[pl.ds(ebase, CHUNK)], didx)
      pltpu.sync_copy(x_hbm.at[sidx], rows)
      plsc.subcore_barrier()

      @pl.loop(0, iters)
      def _edges(i):
        pltpu.sync_copy(rows, acc.at[didx], add=True)
        o = ebase + jnp.minimum(i + 1, iters - 1) * CHUNK
        pltpu.sync_copy(src_hbm.at[pl.ds(o, CHUNK)], sidx)
        pltpu.sync_copy(dst_hbm.at[pl.ds(o, CHUNK)], didx)
        pltpu.sync_copy(x_hbm.at[sidx], rows)

    @pl.when(c == 1)
    def _h_core():
      pltpu.sync_copy(init_hbm, rows)
      pltpu.sync_copy(dst_hbm.at[pl.ds(ebase, CHUNK)], didx)
      pltpu.sync_copy(feat_hbm.at[pl.ds(ebase, CHUNK)], featv)
      for r in range(CHUNK):
        rows[r, pl.ds(0, DE)] = featv[r, :]
      plsc.subcore_barrier()

      @pl.loop(0, iters)
      def _edges(i):
        pltpu.sync_copy(rows, acc.at[didx], add=True)
        o = ebase + jnp.minimum(i + 1, iters - 1) * CHUNK
        pltpu.sync_copy(dst_hbm.at[pl.ds(o, CHUNK)], didx)
        pltpu.sync_copy(feat_hbm.at[pl.ds(o, CHUNK)], featv)
        for r in range(CHUNK):
          rows[r, pl.ds(0, DE)] = featv[r, :]

    plsc.subcore_barrier()

    @pl.when(c == 0)
    def _outg():
      pltpu.sync_copy(acc.at[pl.ds(base_r, rows_per_tile)],
                      outg.at[pl.ds(base_r, rows_per_tile)])

    @pl.when(c == 1)
    def _outh():
      pltpu.sync_copy(acc.at[pl.ds(base_r, rows_per_tile)],
                      outh.at[pl.ds(base_r, rows_per_tile)])

  return pl.kernel(body, out_type=out_type, mesh=mesh,
                   scratch_types=scratch)


def _xnew_body(g, hx, w, wr, ox):
  hxv = hx[...]
  deg = hxv[:, DE]
  agg = (jnp.dot(g[...], w[...], preferred_element_type=jnp.float32)
         + jnp.dot(hxv[:, :DE], wr[...], preferred_element_type=jnp.float32))
  ox[...] = jnp.maximum(agg / jnp.clip(deg, 1.0, None)[:, None], 0.0)


@functools.lru_cache(maxsize=None)
def _make_xnew(npad, blk=512):
  nblk = npad // blk
  wide = pl.BlockSpec((blk, D), lambda i: (i, 0))
  return pl.pallas_call(
      _xnew_body,
      grid=(nblk,),
      in_specs=[wide, wide,
                pl.BlockSpec((D, D), lambda i: (0, 0)),
                pl.BlockSpec((DE, D), lambda i: (0, 0))],
      out_specs=wide,
      out_shape=jax.ShapeDtypeStruct((npad, D), jnp.float32),
  )


def _edge_body(e0, b1, b2, o1, o2):
  e1 = jnp.maximum(jnp.dot(e0[...], b1[...],
                           preferred_element_type=jnp.float32), 0.0)
  o1[...] = e1
  o2[...] = jnp.maximum(jnp.dot(e1, b2[...],
                                preferred_element_type=jnp.float32), 0.0)


@functools.lru_cache(maxsize=None)
def _make_edge_pass(nrows, blk=2000):
  spec = pl.BlockSpec((blk, D), lambda i: (i, 0))
  wspec = pl.BlockSpec((D, D), lambda i: (0, 0))
  return pl.pallas_call(
      _edge_body,
      grid=(nrows // blk,),
      in_specs=[spec, wspec, wspec],
      out_specs=[spec, spec],
      out_shape=[jax.ShapeDtypeStruct((nrows, D), jnp.float32),
                 jax.ShapeDtypeStruct((nrows, D), jnp.float32)],
  )


def kernel(x, edge_index, edge_attr, W1, Wr1, We1, W2, Wr2, We2):
  n, d = x.shape
  e = edge_attr.shape[0]
  npad = ((n + NS * 16 - 1) // (NS * 16)) * (NS * 16)

  src = edge_index[0]
  dst = edge_index[1]

  eye8 = jnp.eye(8, dtype=jnp.float32)
  b1 = jnp.kron(eye8, We1)
  b2 = jnp.kron(eye8, We2)
  e0r = edge_attr.reshape(e // 8, 8 * DE)
  e1r, e2r = _make_edge_pass(e // 8)(e0r, b1, b2)
  e1f = e1r.reshape(e, DE)

  init_rows = jnp.zeros((CHUNK, D), jnp.float32).at[:, DE].set(1.0)

  g1, hx1 = _make_sc_pass(n, e, npad)(x, edge_attr, src, dst, init_rows)
  x1 = _make_xnew(npad)(g1, hx1, W1, Wr1)

  g2, hx2 = _make_sc_pass(npad, e, npad)(x1, e1f, src, dst, init_rows)
  x2 = _make_xnew(npad)(g2, hx2, W2, Wr2)

  return x2[:n], e2r.reshape(e, DE)


# speedup vs baseline: 3.9233x; 1.5269x over previous
import functools

import jax
import jax.numpy as jnp
from jax import lax
from jax.experimental import pallas as pl
from jax.experimental.pallas import tpu as pltpu
from jax.experimental.pallas import tpu_sc as plsc

NC = 2
NS = 16
D = 128
DE = 16
CHUNK = 80


@functools.lru_cache(maxsize=None)
def _make_sc_pass(n_table, n_edges, npad):
  per_tile = n_edges // NS
  iters = per_tile // CHUNK
  assert per_tile % CHUNK == 0
  rows_per_tile = npad // NS

  mesh = plsc.VectorSubcoreMesh(core_axis_name="c", subcore_axis_name="s")

  out_type = [
      jax.ShapeDtypeStruct((npad, D), jnp.float32),
      jax.ShapeDtypeStruct((npad, D), jnp.float32),
  ]
  assert iters % 2 == 0
  scratch = [
      pltpu.VMEM((CHUNK,), jnp.int32),
      pltpu.VMEM((CHUNK,), jnp.int32),
      pltpu.VMEM((CHUNK,), jnp.int32),
      pltpu.VMEM((CHUNK,), jnp.int32),
      pltpu.VMEM((CHUNK, D), jnp.float32),
      pltpu.VMEM((CHUNK, D), jnp.float32),
      pltpu.VMEM((CHUNK, DE), jnp.float32),
      pltpu.VMEM((CHUNK, DE), jnp.float32),
      pltpu.VMEM((16, D), jnp.float32),
      pltpu.VMEM_SHARED((npad, D), jnp.float32),
      pltpu.SemaphoreType.DMA,
      pltpu.SemaphoreType.DMA,
  ]

  def body(x_hbm, feat_hbm, src_hbm, dst_hbm, init_hbm, outg, outh,
           sidx_a, sidx_b, didx_a, didx_b, rows_a, rows_b,
           featv_a, featv_b, zg, acc, sem_a, sem_b):
    c = lax.axis_index("c")
    s = lax.axis_index("s")

    zvec = jnp.zeros((16,), jnp.float32)
    for r in range(16):
      for q in range(D // 16):
        zg[r, pl.ds(q * 16, 16)] = zvec

    base_r = s * rows_per_tile

    @pl.loop(0, rows_per_tile // 16)
    def _zero(i):
      pltpu.sync_copy(zg, acc.at[pl.ds(base_r + i * 16, 16)])

    ebase = s * per_tile
    last_o = ebase + (iters - 1) * CHUNK

    @pl.when(c == 0)
    def _g_core():
      pltpu.sync_copy(src_hbm.at[pl.ds(ebase, CHUNK)], sidx_a)
      pltpu.sync_copy(dst_hbm.at[pl.ds(ebase, CHUNK)], didx_a)
      pltpu.sync_copy(x_hbm.at[sidx_a], rows_a)
      pltpu.sync_copy(src_hbm.at[pl.ds(ebase + CHUNK, CHUNK)], sidx_b)
      pltpu.sync_copy(dst_hbm.at[pl.ds(ebase + CHUNK, CHUNK)], didx_b)
      plsc.subcore_barrier()

      @pl.loop(0, iters // 2)
      def _pairs(k):
        o2k = ebase + 2 * k * CHUNK
        cgb = pltpu.async_copy(x_hbm.at[sidx_b], rows_b, sem_b)
        pltpu.sync_copy(rows_a, acc.at[didx_a], add=True)
        oa = jnp.minimum(o2k + 2 * CHUNK, last_o)
        ca1 = pltpu.async_copy(src_hbm.at[pl.ds(oa, CHUNK)], sidx_a, sem_a)
        ca2 = pltpu.async_copy(dst_hbm.at[pl.ds(oa, CHUNK)], didx_a, sem_a)
        cgb.wait()
        ca1.wait()
        ca2.wait()
        cga = pltpu.async_copy(x_hbm.at[sidx_a], rows_a, sem_a)
        pltpu.sync_copy(rows_b, acc.at[didx_b], add=True)
        ob = jnp.minimum(o2k + 3 * CHUNK, last_o)
        cb1 = pltpu.async_copy(src_hbm.at[pl.ds(ob, CHUNK)], sidx_b, sem_b)
        cb2 = pltpu.async_copy(dst_hbm.at[pl.ds(ob, CHUNK)], didx_b, sem_b)
        cga.wait()
        cb1.wait()
        cb2.wait()

    @pl.when(c == 1)
    def _h_core():
      pltpu.sync_copy(init_hbm, rows_a)
      pltpu.sync_copy(init_hbm, rows_b)

      def relay(fv, rw):
        for r in range(CHUNK):
          rw[r, pl.ds(0, DE)] = fv[r, :]

      pltpu.sync_copy(dst_hbm.at[pl.ds(ebase, CHUNK)], didx_a)
      pltpu.sync_copy(feat_hbm.at[pl.ds(ebase, CHUNK)], featv_a)
      relay(featv_a, rows_a)
      pltpu.sync_copy(dst_hbm.at[pl.ds(ebase + CHUNK, CHUNK)], didx_b)
      pltpu.sync_copy(feat_hbm.at[pl.ds(ebase + CHUNK, CHUNK)], featv_b)
      plsc.subcore_barrier()

      @pl.loop(0, iters // 2)
      def _pairs(k):
        o2k = ebase + 2 * k * CHUNK
        pltpu.sync_copy(rows_a, acc.at[didx_a], add=True)
        relay(featv_b, rows_b)
        oa = jnp.minimum(o2k + 2 * CHUNK, last_o)
        ca1 = pltpu.async_copy(dst_hbm.at[pl.ds(oa, CHUNK)], didx_a, sem_a)
        ca2 = pltpu.async_copy(feat_hbm.at[pl.ds(oa, CHUNK)], featv_a, sem_a)
        pltpu.sync_copy(rows_b, acc.at[didx_b], add=True)
        ca1.wait()
        ca2.wait()
        relay(featv_a, rows_a)
        ob = jnp.minimum(o2k + 3 * CHUNK, last_o)
        cb1 = pltpu.async_copy(dst_hbm.at[pl.ds(ob, CHUNK)], didx_b, sem_b)
        cb2 = pltpu.async_copy(feat_hbm.at[pl.ds(ob, CHUNK)], featv_b, sem_b)
        cb1.wait()
        cb2.wait()

    plsc.subcore_barrier()

    @pl.when(c == 0)
    def _outg():
      pltpu.sync_copy(acc.at[pl.ds(base_r, rows_per_tile)],
                      outg.at[pl.ds(base_r, rows_per_tile)])

    @pl.when(c == 1)
    def _outh():
      pltpu.sync_copy(acc.at[pl.ds(base_r, rows_per_tile)],
                      outh.at[pl.ds(base_r, rows_per_tile)])

  return pl.kernel(body, out_type=out_type, mesh=mesh,
                   scratch_types=scratch)


def _xnew_body(g, hx, w, wr, ox):
  hxv = hx[...]
  deg = hxv[:, DE]
  agg = (jnp.dot(g[...], w[...], preferred_element_type=jnp.float32)
         + jnp.dot(hxv[:, :DE], wr[...], preferred_element_type=jnp.float32))
  ox[...] = jnp.maximum(agg / jnp.clip(deg, 1.0, None)[:, None], 0.0)


@functools.lru_cache(maxsize=None)
def _make_xnew(npad, blk=512):
  nblk = npad // blk
  wide = pl.BlockSpec((blk, D), lambda i: (i, 0))
  return pl.pallas_call(
      _xnew_body,
      grid=(nblk,),
      in_specs=[wide, wide,
                pl.BlockSpec((D, D), lambda i: (0, 0)),
                pl.BlockSpec((DE, D), lambda i: (0, 0))],
      out_specs=wide,
      out_shape=jax.ShapeDtypeStruct((npad, D), jnp.float32),
  )


def _edge_body(e0, b1, b2, o1, o2):
  e1 = jnp.maximum(jnp.dot(e0[...], b1[...],
                           preferred_element_type=jnp.float32), 0.0)
  o1[...] = e1
  o2[...] = jnp.maximum(jnp.dot(e1, b2[...],
                                preferred_element_type=jnp.float32), 0.0)


@functools.lru_cache(maxsize=None)
def _make_edge_pass(nrows, blk=2000):
  spec = pl.BlockSpec((blk, D), lambda i: (i, 0))
  wspec = pl.BlockSpec((D, D), lambda i: (0, 0))
  return pl.pallas_call(
      _edge_body,
      grid=(nrows // blk,),
      in_specs=[spec, wspec, wspec],
      out_specs=[spec, spec],
      out_shape=[jax.ShapeDtypeStruct((nrows, D), jnp.float32),
                 jax.ShapeDtypeStruct((nrows, D), jnp.float32)],
  )


def kernel(x, edge_index, edge_attr, W1, Wr1, We1, W2, Wr2, We2):
  n, d = x.shape
  e = edge_attr.shape[0]
  npad = ((n + NS * 16 - 1) // (NS * 16)) * (NS * 16)

  src = edge_index[0]
  dst = edge_index[1]

  eye8 = jnp.eye(8, dtype=jnp.float32)
  b1 = jnp.kron(eye8, We1)
  b2 = jnp.kron(eye8, We2)
  e0r = edge_attr.reshape(e // 8, 8 * DE)
  e1r, e2r = _make_edge_pass(e // 8)(e0r, b1, b2)
  e1f = e1r.reshape(e, DE)

  init_rows = jnp.zeros((CHUNK, D), jnp.float32).at[:, DE].set(1.0)

  g1, hx1 = _make_sc_pass(n, e, npad)(x, edge_attr, src, dst, init_rows)
  x1 = _make_xnew(npad)(g1, hx1, W1, Wr1)

  g2, hx2 = _make_sc_pass(npad, e, npad)(x1, e1f, src, dst, init_rows)
  x2 = _make_xnew(npad)(g2, hx2, W2, Wr2)

  return x2[:n], e2r.reshape(e, DE)
